# baseline (speedup 1.0000x reference)
import functools

import jax
import jax.numpy as jnp
from jax import lax
from jax.experimental import pallas as pl
from jax.experimental.pallas import tpu as pltpu
from jax.experimental.pallas import tpu_sc as plsc

N = 10000
E = 320000
F = 128
NC = 2
NS = 16
NW = NC * NS
EPT = E // NW
CHUNK = 80
NCHUNK = EPT // CHUNK
NPH = 5
PH = NCHUNK // NPH
NSLOT = 4
NRW = 10112
SPTW = NRW // NS
NP = 10240
SPT = NP // NS
EPP = 10240
RB = EPP // 128
LPR = 8

_MESH = plsc.VectorSubcoreMesh(core_axis_name="c", subcore_axis_name="s")


def _sc_agg_wide(src_hbm, dst_hbm, table_hbm, zero_hbm, out_hbm,
                 src_buf, dst_buf, rows_buf, acc, sem, sem_s):
    c = lax.axis_index("c")
    s = lax.axis_index("s")
    wid = c * NS + s
    r0 = s * SPTW

    @pl.when(c == 0)
    def _():
        pltpu.sync_copy(table_hbm.at[pl.ds(r0, SPTW)],
                        acc.at[pl.ds(r0, SPTW)])

    @pl.when(c != 0)
    def _():
        pltpu.sync_copy(zero_hbm.at[pl.ds(r0, SPTW)],
                        acc.at[pl.ds(r0, SPTW)])

    plsc.subcore_barrier()

    for phase in range(NPH):
        pltpu.sync_copy(src_hbm.at[wid, phase], src_buf)
        pltpu.sync_copy(dst_hbm.at[wid, phase], dst_buf)
        for p in range(NSLOT - 1):
            pltpu.async_copy(table_hbm.at[src_buf.at[p]],
                             rows_buf.at[p], sem)

        def body(j, carry):
            slot = lax.rem(j, NSLOT)

            @pl.when(j >= 1)
            def _():
                pltpu.make_async_copy(rows_buf.at[slot],
                                      acc.at[dst_buf.at[0]], sem_s).wait()

            @pl.when(j < PH - (NSLOT - 1))
            def _():
                pltpu.async_copy(
                    table_hbm.at[src_buf.at[j + NSLOT - 1]],
                    rows_buf.at[lax.rem(j + NSLOT - 1, NSLOT)], sem)

            pltpu.make_async_copy(table_hbm.at[src_buf.at[j]],
                                  rows_buf.at[slot], sem).wait()
            pltpu.async_copy(rows_buf.at[slot], acc.at[dst_buf.at[j]],
                             sem_s, add=True)
            return carry

        lax.fori_loop(0, PH, body, 0)
        pltpu.make_async_copy(rows_buf.at[0], acc.at[dst_buf.at[0]],
                              sem_s).wait()
    plsc.subcore_barrier()
    pltpu.sync_copy(acc.at[pl.ds(r0, SPTW)],
                    out_hbm.at[c, pl.ds(r0, SPTW)])


_agg_wide = functools.partial(
    pl.kernel,
    out_type=jax.ShapeDtypeStruct((NC, NP, F), jnp.float32),
    mesh=_MESH,
    scratch_types=[
        pltpu.VMEM((PH, CHUNK), jnp.int32),
        pltpu.VMEM((PH, CHUNK), jnp.int32),
        pltpu.VMEM((NSLOT, CHUNK, F), jnp.float32),
        pltpu.VMEM_SHARED((NRW, F), jnp.float32),
        pltpu.SemaphoreType.DMA,
        pltpu.SemaphoreType.DMA,
    ],
)(_sc_agg_wide)


def _sc_deg(dst_hbm, ones_hbm, zero_hbm, out_hbm,
            dst_buf, accv, red_buf, res_buf, slots):
    c = lax.axis_index("c")
    s = lax.axis_index("s")
    wid = c * NS + s
    pltpu.sync_copy(dst_hbm.at[wid], dst_buf)

    @pl.when(jnp.logical_and(c == 0, s == 0))
    def _():
        pltpu.sync_copy(ones_hbm, accv)

    @pl.when(jnp.logical_or(c != 0, s != 0))
    def _():
        pltpu.sync_copy(zero_hbm, accv)

    ones16 = jnp.ones((16,), jnp.float32)

    def body(i, carry):
        for k in range(LPR):
            di = dst_buf[i, pl.ds(k * 16, 16)]
            plsc.addupdate_scatter(accv, [di], ones16)
        return carry

    lax.fori_loop(0, RB, body, 0)
    pltpu.sync_copy(accv, slots.at[s])
    plsc.subcore_barrier()
    r0 = s * SPT
    pltpu.sync_copy(slots.at[:, pl.ds(r0, SPT)], red_buf)

    for k in range(SPT // 16):
        o = k * 16
        tot = red_buf[0, pl.ds(o, 16)]
        for t in range(1, NS):
            tot = tot + red_buf[t, pl.ds(o, 16)]
        res_buf[pl.ds(o, 16)] = tot
    pltpu.sync_copy(res_buf, out_hbm.at[c, pl.ds(r0, SPT)])


_deg = functools.partial(
    pl.kernel,
    out_type=jax.ShapeDtypeStruct((NC, NP), jnp.float32),
    mesh=_MESH,
    scratch_types=[
        pltpu.VMEM((RB, 128), jnp.int32),
        pltpu.VMEM((NP,), jnp.float32),
        pltpu.VMEM((NS, SPT), jnp.float32),
        pltpu.VMEM((SPT,), jnp.float32),
        pltpu.VMEM_SHARED((NS, NP), jnp.float32),
    ],
    compiler_params=pltpu.CompilerParams(needs_layout_passes=False),
)(_sc_deg)


def _sc_agg_scalar(src_hbm, dst_hbm, table_hbm, zero_hbm, out_hbm,
                   src_buf, dst_buf, table_v, accv, red_buf, res_buf, slots):
    c = lax.axis_index("c")
    s = lax.axis_index("s")
    wid = c * NS + s
    pltpu.sync_copy(src_hbm.at[wid], src_buf)
    pltpu.sync_copy(dst_hbm.at[wid], dst_buf)
    pltpu.sync_copy(table_hbm, table_v)

    @pl.when(jnp.logical_and(c == 0, s == 0))
    def _():
        pltpu.sync_copy(table_hbm, accv)

    @pl.when(jnp.logical_or(c != 0, s != 0))
    def _():
        pltpu.sync_copy(zero_hbm, accv)

    def body(i, carry):
        for k in range(LPR):
            si = src_buf[i, pl.ds(k * 16, 16)]
            di = dst_buf[i, pl.ds(k * 16, 16)]
            v = plsc.load_gather(table_v, [si])
            plsc.addupdate_scatter(accv, [di], v)
        return carry

    lax.fori_loop(0, RB, body, 0)
    pltpu.sync_copy(accv, slots.at[s])
    plsc.subcore_barrier()
    r0 = s * SPT
    pltpu.sync_copy(slots.at[:, pl.ds(r0, SPT)], red_buf)

    for k in range(SPT // 16):
        o = k * 16
        tot = red_buf[0, pl.ds(o, 16)]
        for t in range(1, NS):
            tot = tot + red_buf[t, pl.ds(o, 16)]
        res_buf[pl.ds(o, 16)] = tot
    pltpu.sync_copy(res_buf, out_hbm.at[c, pl.ds(r0, SPT)])


_agg_scalar = functools.partial(
    pl.kernel,
    out_type=jax.ShapeDtypeStruct((NC, NP), jnp.float32),
    mesh=_MESH,
    scratch_types=[
        pltpu.VMEM((RB, 128), jnp.int32),
        pltpu.VMEM((RB, 128), jnp.int32),
        pltpu.VMEM((NP,), jnp.float32),
        pltpu.VMEM((NP,), jnp.float32),
        pltpu.VMEM((NS, SPT), jnp.float32),
        pltpu.VMEM((SPT,), jnp.float32),
        pltpu.VMEM_SHARED((NS, NP), jnp.float32),
    ],
    compiler_params=pltpu.CompilerParams(needs_layout_passes=False),
)(_sc_agg_scalar)


BR = 1024
_GRID = NP // BR


def _dis(degp_ref):
    deg = degp_ref[:, 0:1] + degp_ref[:, 1:2]
    return lax.rsqrt(deg)


def _tc_l1(x_ref, w_ref, degp_ref, out_ref):
    h = jnp.dot(x_ref[...], w_ref[...], preferred_element_type=jnp.float32)
    out_ref[...] = h * _dis(degp_ref)


def _tc_mid(p0_ref, p1_ref, degp_ref, b_ref, w_ref, out_ref):
    dis = _dis(degp_ref)
    xin = jnp.maximum(dis * (p0_ref[0] + p1_ref[0]) + b_ref[...], 0.0)
    out_ref[...] = dis * jnp.dot(xin, w_ref[...],
                                 preferred_element_type=jnp.float32)


def _tc_fin(p3_ref, degp_ref, b_ref, out_ref):
    dis = _dis(degp_ref)
    z = dis * (p3_ref[:, 0:1] + p3_ref[:, 1:2]) + b_ref[...]
    out_ref[...] = jax.nn.sigmoid(z)


def _l1_call(x, w1, degp):
    return pl.pallas_call(
        _tc_l1,
        grid=(_GRID,),
        in_specs=[
            pl.BlockSpec((BR, F), lambda i: (i, 0)),
            pl.BlockSpec((F, F), lambda i: (0, 0)),
            pl.BlockSpec((BR, 2), lambda i: (i, 0)),
        ],
        out_specs=pl.BlockSpec((BR, F), lambda i: (i, 0)),
        out_shape=jax.ShapeDtypeStruct((NP, F), jnp.float32),
    )(x, w1, degp)


def _mid_call(p, degp, b, w, fout):
    return pl.pallas_call(
        _tc_mid,
        grid=(_GRID,),
        in_specs=[
            pl.BlockSpec((1, BR, F), lambda i: (0, i, 0)),
            pl.BlockSpec((1, BR, F), lambda i: (1, i, 0)),
            pl.BlockSpec((BR, 2), lambda i: (i, 0)),
            pl.BlockSpec((1, F), lambda i: (0, 0)),
            pl.BlockSpec((F, fout), lambda i: (0, 0)),
        ],
        out_specs=pl.BlockSpec((BR, fout), lambda i: (i, 0)),
        out_shape=jax.ShapeDtypeStruct((NP, fout), jnp.float32),
    )(p, p, degp, b, w)


def _fin_call(p3t, degp, b3):
    return pl.pallas_call(
        _tc_fin,
        grid=(_GRID,),
        in_specs=[
            pl.BlockSpec((BR, 2), lambda i: (i, 0)),
            pl.BlockSpec((BR, 2), lambda i: (i, 0)),
            pl.BlockSpec((1, 1), lambda i: (0, 0)),
        ],
        out_specs=pl.BlockSpec((BR, 1), lambda i: (i, 0)),
        out_shape=jax.ShapeDtypeStruct((NP, 1), jnp.float32),
    )(p3t, degp, b3)


def kernel(x, edge_index, W1, b1, W2, b2, W3, b3):
    src2d = edge_index[0].reshape(NW, NPH, PH, CHUNK)
    dst2d = edge_index[1].reshape(NW, NPH, PH, CHUNK)
    epad = jnp.full((NW * EPP - E,), NP - 1, jnp.int32)
    srcp = jnp.concatenate([edge_index[0], epad]).reshape(NW, RB, 128)
    dstp = jnp.concatenate([edge_index[1], epad]).reshape(NW, RB, 128)
    xp = jnp.pad(x, ((0, NP - N), (0, 0)))
    onesP = jnp.ones((NP,), jnp.float32)
    zeroP = jnp.zeros((NP,), jnp.float32)
    zeroF = jnp.zeros((NP, F), jnp.float32)
    b1r = b1.reshape(1, F)
    b2r = b2.reshape(1, F)
    b3r = b3.reshape(1, 1)

    degp2 = _deg(dstp, onesP, zeroP)
    degp = degp2.T

    h1s = _l1_call(xp, W1, degp)
    p1 = _agg_wide(src2d, dst2d, h1s, zeroF)
    h2s = _mid_call(p1, degp, b1r, W2, F)
    p2 = _agg_wide(src2d, dst2d, h2s, zeroF)
    h3s = _mid_call(p2, degp, b2r, W3, 1)
    p3 = _agg_scalar(srcp, dstp, h3s[:, 0], zeroP)
    p3t = p3.T
    return _fin_call(p3t, degp, b3r)[:N]


# speedup vs baseline: 1.0141x; 1.0141x over previous
import functools

import jax
import jax.numpy as jnp
from jax import lax
from jax.experimental import pallas as pl
from jax.experimental.pallas import tpu as pltpu
from jax.experimental.pallas import tpu_sc as plsc

N = 10000
E = 320000
F = 128
NC = 2
NS = 16
NW = NC * NS
EPT = E // NW
CHUNK = 100
NCHUNK = EPT // CHUNK
NPH = 4
PH = NCHUNK // NPH
NSLOT = 3
NP = 10240
SPT = NP // NS
EPP = 10240
RB = EPP // 128
LPR = 8

_MESH = plsc.VectorSubcoreMesh(core_axis_name="c", subcore_axis_name="s")


def _sc_agg_wide(src_hbm, dst_hbm, table_hbm, zero_hbm, out_hbm,
                 src_buf, dst_buf, rows_buf, acc, sem, sem_s):
    c = lax.axis_index("c")
    s = lax.axis_index("s")
    wid = c * NS + s
    r0 = s * SPT

    @pl.when(c == 0)
    def _():
        pltpu.sync_copy(table_hbm.at[pl.ds(r0, SPT)],
                        acc.at[pl.ds(r0, SPT)])

    @pl.when(c != 0)
    def _():
        pltpu.sync_copy(zero_hbm.at[pl.ds(r0, SPT)],
                        acc.at[pl.ds(r0, SPT)])

    pltpu.sync_copy(src_hbm.at[wid, 0], src_buf)
    pltpu.sync_copy(dst_hbm.at[wid, 0], dst_buf)
    pltpu.async_copy(table_hbm.at[src_buf.at[0]], rows_buf.at[0], sem)
    pltpu.async_copy(table_hbm.at[src_buf.at[1]], rows_buf.at[1], sem)
    plsc.subcore_barrier()

    for phase in range(NPH):
        if phase > 0:
            pltpu.sync_copy(src_hbm.at[wid, phase], src_buf)
            pltpu.sync_copy(dst_hbm.at[wid, phase], dst_buf)
            pltpu.async_copy(table_hbm.at[src_buf.at[0]], rows_buf.at[0],
                             sem)
            pltpu.async_copy(table_hbm.at[src_buf.at[1]], rows_buf.at[1],
                             sem)

        def body(j, carry):
            slot = lax.rem(j, NSLOT)

            @pl.when(j >= 1)
            def _():
                pltpu.make_async_copy(rows_buf.at[slot],
                                      acc.at[dst_buf.at[0]], sem_s).wait()

            @pl.when(j < PH - 2)
            def _():
                pltpu.async_copy(table_hbm.at[src_buf.at[j + 2]],
                                 rows_buf.at[lax.rem(j + 2, NSLOT)], sem)

            pltpu.make_async_copy(table_hbm.at[src_buf.at[j]],
                                  rows_buf.at[slot], sem).wait()
            pltpu.async_copy(rows_buf.at[slot], acc.at[dst_buf.at[j]],
                             sem_s, add=True)
            return carry

        lax.fori_loop(0, PH, body, 0)
        pltpu.make_async_copy(rows_buf.at[0], acc.at[dst_buf.at[0]],
                              sem_s).wait()
    plsc.subcore_barrier()
    pltpu.sync_copy(acc.at[pl.ds(r0, SPT)],
                    out_hbm.at[c, pl.ds(r0, SPT)])


_agg_wide = functools.partial(
    pl.kernel,
    out_type=jax.ShapeDtypeStruct((NC, NP, F), jnp.float32),
    mesh=_MESH,
    scratch_types=[
        pltpu.VMEM((PH, CHUNK), jnp.int32),
        pltpu.VMEM((PH, CHUNK), jnp.int32),
        pltpu.VMEM((NSLOT, CHUNK, F), jnp.float32),
        pltpu.VMEM_SHARED((NP, F), jnp.float32),
        pltpu.SemaphoreType.DMA,
        pltpu.SemaphoreType.DMA,
    ],
)(_sc_agg_wide)


def _sc_deg(dst_hbm, ones_hbm, zero_hbm, out_hbm,
            dst_buf, accv, red_buf, res_buf, slots):
    c = lax.axis_index("c")
    s = lax.axis_index("s")
    wid = c * NS + s
    pltpu.sync_copy(dst_hbm.at[wid], dst_buf)

    @pl.when(jnp.logical_and(c == 0, s == 0))
    def _():
        pltpu.sync_copy(ones_hbm, accv)

    @pl.when(jnp.logical_or(c != 0, s != 0))
    def _():
        pltpu.sync_copy(zero_hbm, accv)

    ones16 = jnp.ones((16,), jnp.float32)

    def body(i, carry):
        for k in range(LPR):
            di = dst_buf[i, pl.ds(k * 16, 16)]
            plsc.addupdate_scatter(accv, [di], ones16)
        return carry

    lax.fori_loop(0, RB, body, 0)
    pltpu.sync_copy(accv, slots.at[s])
    plsc.subcore_barrier()
    r0 = s * SPT
    pltpu.sync_copy(slots.at[:, pl.ds(r0, SPT)], red_buf)

    for k in range(SPT // 16):
        o = k * 16
        tot = red_buf[0, pl.ds(o, 16)]
        for t in range(1, NS):
            tot = tot + red_buf[t, pl.ds(o, 16)]
        res_buf[pl.ds(o, 16)] = tot
    pltpu.sync_copy(res_buf, out_hbm.at[c, pl.ds(r0, SPT)])


_deg = functools.partial(
    pl.kernel,
    out_type=jax.ShapeDtypeStruct((NC, NP), jnp.float32),
    mesh=_MESH,
    scratch_types=[
        pltpu.VMEM((RB, 128), jnp.int32),
        pltpu.VMEM((NP,), jnp.float32),
        pltpu.VMEM((NS, SPT), jnp.float32),
        pltpu.VMEM((SPT,), jnp.float32),
        pltpu.VMEM_SHARED((NS, NP), jnp.float32),
    ],
    compiler_params=pltpu.CompilerParams(needs_layout_passes=False),
)(_sc_deg)


def _sc_agg_scalar(src_hbm, dst_hbm, table_hbm, zero_hbm, out_hbm,
                   src_buf, dst_buf, table_v, accv, red_buf, res_buf, slots):
    c = lax.axis_index("c")
    s = lax.axis_index("s")
    wid = c * NS + s
    pltpu.sync_copy(src_hbm.at[wid], src_buf)
    pltpu.sync_copy(dst_hbm.at[wid], dst_buf)
    pltpu.sync_copy(table_hbm, table_v)

    @pl.when(jnp.logical_and(c == 0, s == 0))
    def _():
        pltpu.sync_copy(table_hbm, accv)

    @pl.when(jnp.logical_or(c != 0, s != 0))
    def _():
        pltpu.sync_copy(zero_hbm, accv)

    def body(i, carry):
        for k in range(LPR):
            si = src_buf[i, pl.ds(k * 16, 16)]
            di = dst_buf[i, pl.ds(k * 16, 16)]
            v = plsc.load_gather(table_v, [si])
            plsc.addupdate_scatter(accv, [di], v)
        return carry

    lax.fori_loop(0, RB, body, 0)
    pltpu.sync_copy(accv, slots.at[s])
    plsc.subcore_barrier()
    r0 = s * SPT
    pltpu.sync_copy(slots.at[:, pl.ds(r0, SPT)], red_buf)

    for k in range(SPT // 16):
        o = k * 16
        tot = red_buf[0, pl.ds(o, 16)]
        for t in range(1, NS):
            tot = tot + red_buf[t, pl.ds(o, 16)]
        res_buf[pl.ds(o, 16)] = tot
    pltpu.sync_copy(res_buf, out_hbm.at[c, pl.ds(r0, SPT)])


_agg_scalar = functools.partial(
    pl.kernel,
    out_type=jax.ShapeDtypeStruct((NC, NP), jnp.float32),
    mesh=_MESH,
    scratch_types=[
        pltpu.VMEM((RB, 128), jnp.int32),
        pltpu.VMEM((RB, 128), jnp.int32),
        pltpu.VMEM((NP,), jnp.float32),
        pltpu.VMEM((NP,), jnp.float32),
        pltpu.VMEM((NS, SPT), jnp.float32),
        pltpu.VMEM((SPT,), jnp.float32),
        pltpu.VMEM_SHARED((NS, NP), jnp.float32),
    ],
    compiler_params=pltpu.CompilerParams(needs_layout_passes=False),
)(_sc_agg_scalar)


BR = 1024
_GRID = NP // BR


def _dis(degp_ref):
    deg = degp_ref[:, 0:1] + degp_ref[:, 1:2]
    return lax.rsqrt(deg)


def _tc_l1(x_ref, w_ref, degp_ref, out_ref):
    h = jnp.dot(x_ref[...], w_ref[...], preferred_element_type=jnp.float32)
    out_ref[...] = h * _dis(degp_ref)


def _tc_mid(p0_ref, p1_ref, degp_ref, b_ref, w_ref, out_ref):
    dis = _dis(degp_ref)
    xin = jnp.maximum(dis * (p0_ref[0] + p1_ref[0]) + b_ref[...], 0.0)
    out_ref[...] = dis * jnp.dot(xin, w_ref[...],
                                 preferred_element_type=jnp.float32)


def _tc_fin(p3_ref, degp_ref, b_ref, out_ref):
    dis = _dis(degp_ref)
    z = dis * (p3_ref[:, 0:1] + p3_ref[:, 1:2]) + b_ref[...]
    out_ref[...] = jax.nn.sigmoid(z)


def _l1_call(x, w1, degp):
    return pl.pallas_call(
        _tc_l1,
        grid=(_GRID,),
        in_specs=[
            pl.BlockSpec((BR, F), lambda i: (i, 0)),
            pl.BlockSpec((F, F), lambda i: (0, 0)),
            pl.BlockSpec((BR, 2), lambda i: (i, 0)),
        ],
        out_specs=pl.BlockSpec((BR, F), lambda i: (i, 0)),
        out_shape=jax.ShapeDtypeStruct((NP, F), jnp.float32),
    )(x, w1, degp)


def _mid_call(p, degp, b, w, fout):
    return pl.pallas_call(
        _tc_mid,
        grid=(_GRID,),
        in_specs=[
            pl.BlockSpec((1, BR, F), lambda i: (0, i, 0)),
            pl.BlockSpec((1, BR, F), lambda i: (1, i, 0)),
            pl.BlockSpec((BR, 2), lambda i: (i, 0)),
            pl.BlockSpec((1, F), lambda i: (0, 0)),
            pl.BlockSpec((F, fout), lambda i: (0, 0)),
        ],
        out_specs=pl.BlockSpec((BR, fout), lambda i: (i, 0)),
        out_shape=jax.ShapeDtypeStruct((NP, fout), jnp.float32),
    )(p, p, degp, b, w)


def _fin_call(p3t, degp, b3):
    return pl.pallas_call(
        _tc_fin,
        grid=(_GRID,),
        in_specs=[
            pl.BlockSpec((BR, 2), lambda i: (i, 0)),
            pl.BlockSpec((BR, 2), lambda i: (i, 0)),
            pl.BlockSpec((1, 1), lambda i: (0, 0)),
        ],
        out_specs=pl.BlockSpec((BR, 1), lambda i: (i, 0)),
        out_shape=jax.ShapeDtypeStruct((NP, 1), jnp.float32),
    )(p3t, degp, b3)


def kernel(x, edge_index, W1, b1, W2, b2, W3, b3):
    src2d = edge_index[0].reshape(NW, NPH, PH, CHUNK)
    dst2d = edge_index[1].reshape(NW, NPH, PH, CHUNK)
    epad = jnp.full((NW * EPP - E,), NP - 1, jnp.int32)
    srcp = jnp.concatenate([edge_index[0], epad]).reshape(NW, RB, 128)
    dstp = jnp.concatenate([edge_index[1], epad]).reshape(NW, RB, 128)
    xp = jnp.pad(x, ((0, NP - N), (0, 0)))
    onesP = jnp.ones((NP,), jnp.float32)
    zeroP = jnp.zeros((NP,), jnp.float32)
    zeroF = jnp.zeros((NP, F), jnp.float32)
    b1r = b1.reshape(1, F)
    b2r = b2.reshape(1, F)
    b3r = b3.reshape(1, 1)

    degp2 = _deg(dstp, onesP, zeroP)
    degp = degp2.T

    h1s = _l1_call(xp, W1, degp)
    p1 = _agg_wide(src2d, dst2d, h1s, zeroF)
    h2s = _mid_call(p1, degp, b1r, W2, F)
    p2 = _agg_wide(src2d, dst2d, h2s, zeroF)
    h3s = _mid_call(p2, degp, b2r, W3, 1)
    p3 = _agg_scalar(srcp, dstp, h3s[:, 0], zeroP)
    p3t = p3.T
    return _fin_call(p3t, degp, b3r)[:N]
